# Initial kernel scaffold; baseline (speedup 1.0000x reference)
import jax
import jax.numpy as jnp
from jax.experimental import pallas as pl


def kernel(hyperedge, hyper_node, ve_affiliation, v2e_W0, v2e_b0, v2e_W1, v2e_b1, v2e_W2, v2e_b2, e2v_W0, e2v_b0, e2v_W1, e2v_b1, e2v_W2, e2v_b2):
    raise NotImplementedError("write your pallas kernel here")


# speedup vs baseline: 2.3847x; 2.3847x over previous
import functools

import jax
import jax.numpy as jnp
from jax import lax
from jax.experimental import pallas as pl
from jax.experimental.pallas import tpu as pltpu
from jax.experimental.pallas import tpu_sc as plsc

N_EDGE = 10000
N_NODE = 320000
DIM = 128
NCORES = 2
NSUB = 16
NTILES = NCORES * NSUB
RPT = N_NODE // NTILES
CHUNK = 80
NCH = RPT // CHUNK
NEP = 10240
EPT = NEP // NSUB

_MESH = plsc.VectorSubcoreMesh(core_axis_name="c", subcore_axis_name="s")


def _seg_body(use_ones, src, idx3d, zeros_e, out, idx_v, rows_v, acc):
    cid = lax.axis_index("c")
    sid = lax.axis_index("s")
    wid = sid * NCORES + cid
    pltpu.sync_copy(zeros_e.at[pl.ds(sid * EPT, EPT)], acc.at[pl.ds(sid * EPT, EPT)])
    pltpu.sync_copy(idx3d.at[wid], idx_v)
    if use_ones:
        pltpu.sync_copy(src, rows_v)
    plsc.subcore_barrier()

    def chunk(c, carry):
        if not use_ones:
            pltpu.sync_copy(src.at[pl.ds(wid * RPT + c * CHUNK, CHUNK)], rows_v)
        pltpu.sync_copy(rows_v, acc.at[idx_v.at[c]], add=True)
        return carry

    lax.fori_loop(0, NCH, chunk, 0)
    plsc.subcore_barrier()
    pltpu.sync_copy(acc.at[pl.ds(sid * EPT, EPT)], out.at[cid, pl.ds(sid * EPT, EPT)])


def _make_seg(use_ones):
    return pl.kernel(
        functools.partial(_seg_body, use_ones),
        out_type=jax.ShapeDtypeStruct((NCORES, NEP, DIM), jnp.float32),
        mesh=_MESH,
        scratch_types=[
            pltpu.VMEM((NCH, CHUNK), jnp.int32),
            pltpu.VMEM((CHUNK, DIM), jnp.float32),
            pltpu.VMEM_SHARED((NEP, DIM), jnp.float32),
        ],
    )


_seg_sum = _make_seg(False)
_cnt_sum = _make_seg(True)


def _gather_body(g, idx3d, out, idx_v, rows_v, sem):
    cid = lax.axis_index("c")
    sid = lax.axis_index("s")
    wid = sid * NCORES + cid
    pltpu.sync_copy(idx3d.at[wid], idx_v)

    def chunk(c, carry):
        pltpu.async_copy(g.at[idx_v.at[c]], rows_v, sem).wait()
        pltpu.sync_copy(rows_v, out.at[pl.ds(wid * RPT + c * CHUNK, CHUNK)])
        return carry

    lax.fori_loop(0, NCH, chunk, 0)


_gather = pl.kernel(
    _gather_body,
    out_type=jax.ShapeDtypeStruct((N_NODE, DIM), jnp.float32),
    mesh=_MESH,
    scratch_types=[
        pltpu.VMEM((NCH, CHUNK), jnp.int32),
        pltpu.VMEM((CHUNK, DIM), jnp.float32),
        pltpu.SemaphoreType.DMA,
    ],
)


def _edge_block(he_ref, s0_ref, s1_ref, c0_ref, c1_ref, vwt_ref, vwb_ref,
                vb_ref, ewt_ref, heo_ref, g_ref):
    cnt = jnp.maximum(c0_ref[...] + c1_ref[...], 1.0)
    agg = (s0_ref[...] + s1_ref[...]) / cnt
    he = jnp.maximum(
        jnp.dot(he_ref[...], vwt_ref[...], preferred_element_type=jnp.float32)
        + jnp.dot(agg, vwb_ref[...], preferred_element_type=jnp.float32)
        + vb_ref[...], 0.0)
    heo_ref[...] = he
    g_ref[...] = jnp.dot(he, ewt_ref[...], preferred_element_type=jnp.float32)


def _edge_update(he, sp, cp, vwt, vwb, vb, ewt):
    blk = 2000
    grid = N_EDGE // blk
    return pl.pallas_call(
        _edge_block,
        grid=(grid,),
        in_specs=[
            pl.BlockSpec((blk, DIM), lambda i: (i, 0)),
            pl.BlockSpec((blk, DIM), lambda i: (i, 0)),
            pl.BlockSpec((blk, DIM), lambda i: (i, 0)),
            pl.BlockSpec((blk, DIM), lambda i: (i, 0)),
            pl.BlockSpec((blk, DIM), lambda i: (i, 0)),
            pl.BlockSpec((DIM, DIM), lambda i: (0, 0)),
            pl.BlockSpec((DIM, DIM), lambda i: (0, 0)),
            pl.BlockSpec((1, DIM), lambda i: (0, 0)),
            pl.BlockSpec((DIM, DIM), lambda i: (0, 0)),
        ],
        out_specs=[
            pl.BlockSpec((blk, DIM), lambda i: (i, 0)),
            pl.BlockSpec((blk, DIM), lambda i: (i, 0)),
        ],
        out_shape=[
            jax.ShapeDtypeStruct((N_EDGE, DIM), jnp.float32),
            jax.ShapeDtypeStruct((N_EDGE, DIM), jnp.float32),
        ],
    )(he, sp[0], sp[1], cp[0], cp[1], vwt, vwb, vb, ewt)


def _node_block(ow, hn_ref, gg_ref, ewb_ref, eb_ref, sel_ref, out_ref):
    gg = gg_ref[...]
    if ow != DIM:
        gg = jnp.dot(gg, sel_ref[...], preferred_element_type=jnp.float32)
    out_ref[...] = jnp.maximum(
        jnp.dot(hn_ref[...], ewb_ref[...], preferred_element_type=jnp.float32)
        + gg + eb_ref[...], 0.0)


def _node_update(hn, gg, ewb, eb, sel, ow):
    blk = 2000
    grid = N_NODE // blk
    return pl.pallas_call(
        functools.partial(_node_block, ow),
        grid=(grid,),
        in_specs=[
            pl.BlockSpec((blk, DIM), lambda i: (i, 0)),
            pl.BlockSpec((blk, DIM), lambda i: (i, 0)),
            pl.BlockSpec((DIM, ow), lambda i: (0, 0)),
            pl.BlockSpec((1, ow), lambda i: (0, 0)),
            pl.BlockSpec((DIM, ow), lambda i: (0, 0)),
        ],
        out_specs=pl.BlockSpec((blk, ow), lambda i: (i, 0)),
        out_shape=jax.ShapeDtypeStruct((N_NODE, ow), jnp.float32),
    )(hn, gg, ewb, eb, sel)


def kernel(hyperedge, hyper_node, ve_affiliation,
           v2e_W0, v2e_b0, v2e_W1, v2e_b1, v2e_W2, v2e_b2,
           e2v_W0, e2v_b0, e2v_W1, e2v_b1, e2v_W2, e2v_b2):
    idx = ve_affiliation[0]
    idx3d = idx.reshape(NTILES, NCH, CHUNK)
    zeros_e = jnp.zeros((NEP, DIM), jnp.float32)
    ones_r = jnp.ones((CHUNK, DIM), jnp.float32)

    vW = ((v2e_W0[:DIM], v2e_W0[DIM:], v2e_b0.reshape(1, DIM)),
          (v2e_W1[:DIM], v2e_W1[DIM:], v2e_b1.reshape(1, DIM)),
          (v2e_W2[:DIM], v2e_W2[DIM:], v2e_b2.reshape(1, DIM)))
    eW = ((e2v_W0[:DIM], e2v_W0[DIM:], e2v_b0.reshape(1, DIM)),
          (e2v_W1[:DIM], e2v_W1[DIM:], e2v_b1.reshape(1, DIM)))
    e2t_pad = jnp.pad(e2v_W2[:DIM], ((0, 0), (0, DIM - 1)))
    e2b = e2v_W2[DIM:]
    sel128 = jnp.eye(DIM, dtype=jnp.float32)
    sel1 = jnp.eye(DIM, 1, dtype=jnp.float32)

    he, hn = hyperedge, hyper_node
    cp = _cnt_sum(ones_r, idx3d, zeros_e)[:, :N_EDGE]
    for l in range(3):
        sp = _seg_sum(hn, idx3d, zeros_e)[:, :N_EDGE]
        if l < 2:
            he, g = _edge_update(he, sp, cp, vW[l][0], vW[l][1], vW[l][2],
                                 eW[l][0])
            gg = _gather(g, idx3d)
            hn = _node_update(hn, gg, eW[l][1], eW[l][2], sel128, DIM)
        else:
            he, g = _edge_update(he, sp, cp, vW[l][0], vW[l][1], vW[l][2],
                                 e2t_pad)
            gg = _gather(g, idx3d)
            hn = _node_update(hn, gg, e2b, e2v_b2.reshape(1, 1), sel1, 1)
    return (he, hn)


# speedup vs baseline: 2.7896x; 1.1698x over previous
import functools

import jax
import jax.numpy as jnp
from jax import lax
from jax.experimental import pallas as pl
from jax.experimental.pallas import tpu as pltpu
from jax.experimental.pallas import tpu_sc as plsc

N_EDGE = 10000
N_NODE = 320000
DIM = 128
NCORES = 2
NSUB = 16
NTILES = NCORES * NSUB
RPT = N_NODE // NTILES
CHUNK = 80
NCH = RPT // CHUNK
NEP = 10240
EPT = NEP // NSUB

_MESH = plsc.VectorSubcoreMesh(core_axis_name="c", subcore_axis_name="s")


def _seg_body(use_ones, src, idx3d, zeros_e, out, idx_v, b0, b1, acc, s0, s1):
    cid = lax.axis_index("c")
    sid = lax.axis_index("s")
    wid = sid * NCORES + cid
    pltpu.sync_copy(zeros_e.at[pl.ds(sid * EPT, EPT)], acc.at[pl.ds(sid * EPT, EPT)])
    pltpu.sync_copy(idx3d.at[wid], idx_v)
    if use_ones:
        pltpu.sync_copy(src, b0)
    plsc.subcore_barrier()
    base = wid * RPT

    if use_ones:
        def chunk(c, carry):
            pltpu.sync_copy(b0, acc.at[idx_v.at[c]], add=True)
            return carry

        lax.fori_loop(0, NCH, chunk, 0)
    else:
        def chunk2(i, carry):
            c = 2 * i
            cp0 = pltpu.async_copy(
                src.at[pl.ds(base + c * CHUNK, CHUNK)], b0, s0)
            cp1 = pltpu.async_copy(
                src.at[pl.ds(base + (c + 1) * CHUNK, CHUNK)], b1, s1)
            cp0.wait()
            pltpu.sync_copy(b0, acc.at[idx_v.at[c]], add=True)
            cp1.wait()
            pltpu.sync_copy(b1, acc.at[idx_v.at[c + 1]], add=True)
            return carry

        lax.fori_loop(0, NCH // 2, chunk2, 0)
        last = NCH - 1
        pltpu.async_copy(src.at[pl.ds(base + last * CHUNK, CHUNK)], b0, s0).wait()
        pltpu.sync_copy(b0, acc.at[idx_v.at[last]], add=True)
    plsc.subcore_barrier()
    pltpu.sync_copy(acc.at[pl.ds(sid * EPT, EPT)], out.at[cid, pl.ds(sid * EPT, EPT)])


def _make_seg(use_ones):
    return pl.kernel(
        functools.partial(_seg_body, use_ones),
        out_type=jax.ShapeDtypeStruct((NCORES, NEP, DIM), jnp.float32),
        mesh=_MESH,
        scratch_types=[
            pltpu.VMEM((NCH, CHUNK), jnp.int32),
            pltpu.VMEM((CHUNK, DIM), jnp.float32),
            pltpu.VMEM((CHUNK, DIM), jnp.float32),
            pltpu.VMEM_SHARED((NEP, DIM), jnp.float32),
            pltpu.SemaphoreType.DMA,
            pltpu.SemaphoreType.DMA,
        ],
    )


_seg_sum = _make_seg(False)
_cnt_sum = _make_seg(True)


def _gather_body(g, idx3d, out, idx_v, b0, b1, s0, s1):
    cid = lax.axis_index("c")
    sid = lax.axis_index("s")
    wid = sid * NCORES + cid
    pltpu.sync_copy(idx3d.at[wid], idx_v)
    base = wid * RPT

    def chunk2(i, carry):
        c = 2 * i
        cp0 = pltpu.async_copy(g.at[idx_v.at[c]], b0, s0)
        cp1 = pltpu.async_copy(g.at[idx_v.at[c + 1]], b1, s1)
        cp0.wait()
        pltpu.sync_copy(b0, out.at[pl.ds(base + c * CHUNK, CHUNK)])
        cp1.wait()
        pltpu.sync_copy(b1, out.at[pl.ds(base + (c + 1) * CHUNK, CHUNK)])
        return carry

    lax.fori_loop(0, NCH // 2, chunk2, 0)
    last = NCH - 1
    pltpu.async_copy(g.at[idx_v.at[last]], b0, s0).wait()
    pltpu.sync_copy(b0, out.at[pl.ds(base + last * CHUNK, CHUNK)])


_gather = pl.kernel(
    _gather_body,
    out_type=jax.ShapeDtypeStruct((N_NODE, DIM), jnp.float32),
    mesh=_MESH,
    scratch_types=[
        pltpu.VMEM((NCH, CHUNK), jnp.int32),
        pltpu.VMEM((CHUNK, DIM), jnp.float32),
        pltpu.VMEM((CHUNK, DIM), jnp.float32),
        pltpu.SemaphoreType.DMA,
        pltpu.SemaphoreType.DMA,
    ],
)


def _edge_block(he_ref, s0_ref, s1_ref, c0_ref, c1_ref, vwt_ref, vwb_ref,
                vb_ref, ewt_ref, heo_ref, g_ref):
    cnt = jnp.maximum(c0_ref[...] + c1_ref[...], 1.0)
    agg = (s0_ref[...] + s1_ref[...]) / cnt
    he = jnp.maximum(
        jnp.dot(he_ref[...], vwt_ref[...], preferred_element_type=jnp.float32)
        + jnp.dot(agg, vwb_ref[...], preferred_element_type=jnp.float32)
        + vb_ref[...], 0.0)
    heo_ref[...] = he
    g_ref[...] = jnp.dot(he, ewt_ref[...], preferred_element_type=jnp.float32)


def _edge_update(he, sp, cp, vwt, vwb, vb, ewt):
    blk = 2000
    grid = N_EDGE // blk
    return pl.pallas_call(
        _edge_block,
        grid=(grid,),
        in_specs=[
            pl.BlockSpec((blk, DIM), lambda i: (i, 0)),
            pl.BlockSpec((blk, DIM), lambda i: (i, 0)),
            pl.BlockSpec((blk, DIM), lambda i: (i, 0)),
            pl.BlockSpec((blk, DIM), lambda i: (i, 0)),
            pl.BlockSpec((blk, DIM), lambda i: (i, 0)),
            pl.BlockSpec((DIM, DIM), lambda i: (0, 0)),
            pl.BlockSpec((DIM, DIM), lambda i: (0, 0)),
            pl.BlockSpec((1, DIM), lambda i: (0, 0)),
            pl.BlockSpec((DIM, DIM), lambda i: (0, 0)),
        ],
        out_specs=[
            pl.BlockSpec((blk, DIM), lambda i: (i, 0)),
            pl.BlockSpec((blk, DIM), lambda i: (i, 0)),
        ],
        out_shape=[
            jax.ShapeDtypeStruct((N_EDGE, DIM), jnp.float32),
            jax.ShapeDtypeStruct((N_EDGE, DIM), jnp.float32),
        ],
    )(he, sp[0], sp[1], cp[0], cp[1], vwt, vwb, vb, ewt)


def _node_block(ow, hn_ref, gg_ref, ewb_ref, eb_ref, sel_ref, out_ref):
    gg = gg_ref[...]
    if ow != DIM:
        gg = jnp.dot(gg, sel_ref[...], preferred_element_type=jnp.float32)
    out_ref[...] = jnp.maximum(
        jnp.dot(hn_ref[...], ewb_ref[...], preferred_element_type=jnp.float32)
        + gg + eb_ref[...], 0.0)


def _node_update(hn, gg, ewb, eb, sel, ow):
    blk = 2000
    grid = N_NODE // blk
    return pl.pallas_call(
        functools.partial(_node_block, ow),
        grid=(grid,),
        in_specs=[
            pl.BlockSpec((blk, DIM), lambda i: (i, 0)),
            pl.BlockSpec((blk, DIM), lambda i: (i, 0)),
            pl.BlockSpec((DIM, ow), lambda i: (0, 0)),
            pl.BlockSpec((1, ow), lambda i: (0, 0)),
            pl.BlockSpec((DIM, ow), lambda i: (0, 0)),
        ],
        out_specs=pl.BlockSpec((blk, ow), lambda i: (i, 0)),
        out_shape=jax.ShapeDtypeStruct((N_NODE, ow), jnp.float32),
    )(hn, gg, ewb, eb, sel)


def kernel(hyperedge, hyper_node, ve_affiliation,
           v2e_W0, v2e_b0, v2e_W1, v2e_b1, v2e_W2, v2e_b2,
           e2v_W0, e2v_b0, e2v_W1, e2v_b1, e2v_W2, e2v_b2):
    idx = ve_affiliation[0]
    idx3d = idx.reshape(NTILES, NCH, CHUNK)
    zeros_e = jnp.zeros((NEP, DIM), jnp.float32)
    ones_r = jnp.ones((CHUNK, DIM), jnp.float32)

    vW = ((v2e_W0[:DIM], v2e_W0[DIM:], v2e_b0.reshape(1, DIM)),
          (v2e_W1[:DIM], v2e_W1[DIM:], v2e_b1.reshape(1, DIM)),
          (v2e_W2[:DIM], v2e_W2[DIM:], v2e_b2.reshape(1, DIM)))
    eW = ((e2v_W0[:DIM], e2v_W0[DIM:], e2v_b0.reshape(1, DIM)),
          (e2v_W1[:DIM], e2v_W1[DIM:], e2v_b1.reshape(1, DIM)))
    e2t_pad = jnp.pad(e2v_W2[:DIM], ((0, 0), (0, DIM - 1)))
    e2b = e2v_W2[DIM:]
    sel128 = jnp.eye(DIM, dtype=jnp.float32)
    sel1 = jnp.eye(DIM, 1, dtype=jnp.float32)

    he, hn = hyperedge, hyper_node
    cp = _cnt_sum(ones_r, idx3d, zeros_e)[:, :N_EDGE]
    for l in range(3):
        sp = _seg_sum(hn, idx3d, zeros_e)[:, :N_EDGE]
        if l < 2:
            he, g = _edge_update(he, sp, cp, vW[l][0], vW[l][1], vW[l][2],
                                 eW[l][0])
            gg = _gather(g, idx3d)
            hn = _node_update(hn, gg, eW[l][1], eW[l][2], sel128, DIM)
        else:
            he, g = _edge_update(he, sp, cp, vW[l][0], vW[l][1], vW[l][2],
                                 e2t_pad)
            gg = _gather(g, idx3d)
            hn = _node_update(hn, gg, e2b, e2v_b2.reshape(1, 1), sel1, 1)
    return (he, hn)
